# baseline (speedup 1.0000x reference)
import functools

import jax
import jax.numpy as jnp
from jax.experimental import pallas as pl
from jax.experimental.pallas import tpu as pltpu
from jax.experimental.pallas import tpu_sc as plsc

Q = 1024
D = 64
K = 100000
KB = 2048
KPAD = 102400
NBLK = KPAD // KB
R = 32
G = KPAD // R
G4 = G // 4
GB = KB // R
GREAL = K // R
GSEL = 64
CW = GSEL * R
CAND = 50
BISECT_ITERS = 34
PAD_VAL = -2.0
FILL_VAL = -3.0
BIG = 2**30


def _simgm_block(q_ref, k_ref, sim_ref, gm_ref):
    i = pl.program_id(0)
    q = q_ref[...]
    k = k_ref[...]
    qn = q / (jnp.sqrt(jnp.sum(q * q, axis=-1, keepdims=True)) + 1e-12)
    kn = k / (jnp.sqrt(jnp.sum(k * k, axis=-1, keepdims=True)) + 1e-12)
    sim = jax.lax.dot_general(
        qn, kn, (((1,), (1,)), ((), ())),
        preferred_element_type=jnp.float32)

    @pl.when(i == NBLK - 1)
    def _():
        col = jax.lax.broadcasted_iota(jnp.int32, (Q, KB), 1) + i * KB
        sim_ref[...] = jnp.where(col < K, sim, PAD_VAL)

    @pl.when(i < NBLK - 1)
    def _():
        sim_ref[...] = sim

    bgm = jnp.max(sim.reshape(Q, GB, R), axis=-1)
    gid = jax.lax.broadcasted_iota(jnp.int32, (Q, GB), 1) + i * GB
    gm_ref[0] = jnp.where(gid < GREAL, bgm, PAD_VAL)


def _bisect_select(gm_ref, gsel_ref):
    gm = gm_ref[...]

    def bis(_, lohi):
        lo, hi = lohi
        mid = 0.5 * (lo + hi)
        ge = (gm >= mid[None, :, None]).astype(jnp.float32)
        cnt = jnp.sum(jnp.sum(ge, axis=-1), axis=0)
        take = cnt >= float(CAND)
        return jnp.where(take, mid, lo), jnp.where(take, hi, mid)

    lo0 = jnp.full((Q,), -1.01, jnp.float32)
    hi0 = jnp.full((Q,), 1.01, jnp.float32)
    tau, _ = jax.lax.fori_loop(0, BISECT_ITERS, bis, (lo0, hi0))

    gid3 = (jax.lax.broadcasted_iota(jnp.int32, (NBLK, Q, GB), 0) * GB
            + jax.lax.broadcasted_iota(jnp.int32, (NBLK, Q, GB), 2))
    work = jnp.where(gm >= tau[None, :, None], gid3, BIG)
    ocol = jax.lax.broadcasted_iota(jnp.int32, (Q, GSEL), 1)

    def ext(t, carry):
        work, out = carry
        p = jnp.min(jnp.min(work, axis=-1), axis=0)
        work = jnp.where(work == p[None, :, None], BIG, work)
        out = jnp.where(ocol == t, jnp.minimum(p, G - 1)[:, None], out)
        return work, out

    _, gsel = jax.lax.fori_loop(
        0, GSEL, ext, (work, jnp.zeros((Q, GSEL), jnp.int32)))
    gsel_ref[...] = gsel


def _expand(gsel_ref, ridx_ref, sub_ref, civ_ref):
    gsel = gsel_ref[...]
    qrow = jax.lax.broadcasted_iota(jnp.int32, (Q, GSEL), 0)
    ridx_ref[...] = qrow * G4 + gsel // 4
    sub = gsel % 4
    sub_ref[...] = jnp.broadcast_to(
        sub[:, :, None], (Q, GSEL, 16)).reshape(Q, GSEL * 16)
    c32 = jax.lax.broadcasted_iota(jnp.int32, (Q, GSEL, R), 2)
    civ_ref[...] = (gsel[:, :, None] * R + c32).reshape(Q, CW)


def _extract_topk(vals_ref, idx_ref, ovals_ref, oidx_ref):
    v = vals_ref[...]
    ci = idx_ref[...]
    ocol = jax.lax.broadcasted_iota(jnp.int32, (Q, CAND), 1)

    def body(t, carry):
        v, ov, oi = carry
        m = jnp.max(v, axis=1, keepdims=True)
        g = jnp.min(jnp.where(v == m, ci, BIG), axis=1, keepdims=True)
        v = jnp.where(ci == g, FILL_VAL, v)
        ov = jnp.where(ocol == t, m, ov)
        oi = jnp.where(ocol == t, g, oi)
        return v, ov, oi

    ov0 = jnp.zeros((Q, CAND), jnp.float32)
    oi0 = jnp.zeros((Q, CAND), jnp.int32)
    _, ov, oi = jax.lax.fori_loop(0, CAND, body, (v, ov0, oi0))
    ovals_ref[...] = ov
    oidx_ref[...] = oi


NWORK = 32
QW = Q // NWORK


def _sc_gather_body(ridx_hbm, sub_hbm, sim4_hbm, cvals_hbm,
                    ridx_v, sub_v, rows_v, cv_v, sem):
    wid = jax.lax.axis_index("s") * 2 + jax.lax.axis_index("c")
    q0 = wid * QW

    def per_query(ql, carry):
        q = q0 + ql
        pltpu.sync_copy(ridx_hbm.at[q], ridx_v)
        pltpu.sync_copy(sub_hbm.at[q], sub_v)
        pltpu.async_copy(sim4_hbm.at[ridx_v], rows_v, sem).wait()
        for j in range(GSEL):
            s = sub_v[pl.ds(j * 16, 16)]
            e0 = s == 0
            e1 = s == 1
            e2 = s == 2
            r0 = rows_v[j, pl.ds(0, 16)]
            r2 = rows_v[j, pl.ds(32, 16)]
            r4 = rows_v[j, pl.ds(64, 16)]
            r6 = rows_v[j, pl.ds(96, 16)]
            lo = jnp.where(e0, r0, jnp.where(e1, r2, jnp.where(e2, r4, r6)))
            r1 = rows_v[j, pl.ds(16, 16)]
            r3 = rows_v[j, pl.ds(48, 16)]
            r5 = rows_v[j, pl.ds(80, 16)]
            r7 = rows_v[j, pl.ds(112, 16)]
            hi = jnp.where(e0, r1, jnp.where(e1, r3, jnp.where(e2, r5, r7)))
            cv_v[pl.ds(j * R, 16)] = lo
            cv_v[pl.ds(j * R + 16, 16)] = hi
        pltpu.sync_copy(cv_v, cvals_hbm.at[q])
        return carry

    jax.lax.fori_loop(0, QW, per_query, jnp.int32(0))


def _stage_b_sc(ridx, sub, sim4):
    mesh = plsc.VectorSubcoreMesh(core_axis_name="c", subcore_axis_name="s")
    run = functools.partial(
        pl.kernel,
        mesh=mesh,
        out_type=jax.ShapeDtypeStruct((Q, CW), jnp.float32),
        scratch_types=[
            pltpu.VMEM((GSEL,), jnp.int32),
            pltpu.VMEM((GSEL * 16,), jnp.int32),
            pltpu.VMEM((GSEL, 128), jnp.float32),
            pltpu.VMEM((CW,), jnp.float32),
            pltpu.SemaphoreType.DMA,
        ],
    )(_sc_gather_body)
    return run(ridx, sub, sim4)


_INTERPRET = False


def _stage_a(queries, keys_p):
    return pl.pallas_call(
        _simgm_block,
        grid=(NBLK,),
        in_specs=[
            pl.BlockSpec((Q, D), lambda i: (0, 0)),
            pl.BlockSpec((KB, D), lambda i: (i, 0)),
        ],
        out_specs=[
            pl.BlockSpec((Q, KB), lambda i: (0, i)),
            pl.BlockSpec((1, Q, GB), lambda i: (i, 0, 0)),
        ],
        out_shape=[
            jax.ShapeDtypeStruct((Q, KPAD), jnp.float32),
            jax.ShapeDtypeStruct((NBLK, Q, GB), jnp.float32),
        ],
        interpret=_INTERPRET,
    )(queries, keys_p)


def _stage_a2(gm):
    return pl.pallas_call(
        _bisect_select,
        in_specs=[pl.BlockSpec((NBLK, Q, GB), lambda: (0, 0, 0))],
        out_specs=pl.BlockSpec((Q, GSEL), lambda: (0, 0)),
        out_shape=jax.ShapeDtypeStruct((Q, GSEL), jnp.int32),
        interpret=_INTERPRET,
    )(gm)


def _stage_a3(gsel):
    return pl.pallas_call(
        _expand,
        in_specs=[pl.BlockSpec((Q, GSEL), lambda: (0, 0))],
        out_specs=[
            pl.BlockSpec((Q, GSEL), lambda: (0, 0)),
            pl.BlockSpec((Q, GSEL * 16), lambda: (0, 0)),
            pl.BlockSpec((Q, CW), lambda: (0, 0)),
        ],
        out_shape=[
            jax.ShapeDtypeStruct((Q, GSEL), jnp.int32),
            jax.ShapeDtypeStruct((Q, GSEL * 16), jnp.int32),
            jax.ShapeDtypeStruct((Q, CW), jnp.int32),
        ],
        interpret=_INTERPRET,
    )(gsel)


def _stage_d(cvals, cidx):
    return pl.pallas_call(
        _extract_topk,
        in_specs=[
            pl.BlockSpec((Q, CW), lambda: (0, 0)),
            pl.BlockSpec((Q, CW), lambda: (0, 0)),
        ],
        out_specs=[
            pl.BlockSpec((Q, CAND), lambda: (0, 0)),
            pl.BlockSpec((Q, CAND), lambda: (0, 0)),
        ],
        out_shape=[
            jax.ShapeDtypeStruct((Q, CAND), jnp.float32),
            jax.ShapeDtypeStruct((Q, CAND), jnp.int32),
        ],
        interpret=_INTERPRET,
    )(cvals, cidx)


def kernel(queries, keys):
    keys_p = jnp.pad(keys, ((0, KPAD - K), (0, 0)))
    sim, gm = _stage_a(queries, keys_p)
    gsel = _stage_a2(gm)
    ridx, sub, civ = _stage_a3(gsel)
    cvals = _stage_b_sc(ridx, sub, sim.reshape(Q * G4, 128))
    vals, idx = _stage_d(cvals, civ)
    return vals, idx


# speedup vs baseline: 1.0710x; 1.0710x over previous
import functools

import jax
import jax.numpy as jnp
from jax.experimental import pallas as pl
from jax.experimental.pallas import tpu as pltpu
from jax.experimental.pallas import tpu_sc as plsc

Q = 1024
D = 64
K = 100000
KB = 2048
KPAD = 102400
NBLK = KPAD // KB
R = 32
G = KPAD // R
G4 = G // 4
GB = KB // R
GREAL = K // R
GSEL = 56
CW = GSEL * R
CAND = 50
BISECT_ITERS = 34
PAD_VAL = -2.0
FILL_VAL = -3.0
BIG = 2**30


def _simgm_block(q_ref, k_ref, sim_ref, gm_ref):
    i = pl.program_id(0)
    q = q_ref[...]
    k = k_ref[...]
    qn = q / (jnp.sqrt(jnp.sum(q * q, axis=-1, keepdims=True)) + 1e-12)
    kn = k / (jnp.sqrt(jnp.sum(k * k, axis=-1, keepdims=True)) + 1e-12)
    sim = jax.lax.dot_general(
        qn, kn, (((1,), (1,)), ((), ())),
        preferred_element_type=jnp.float32)

    @pl.when(i == NBLK - 1)
    def _():
        col = jax.lax.broadcasted_iota(jnp.int32, (Q, KB), 1) + i * KB
        sim_ref[...] = jnp.where(col < K, sim, PAD_VAL)

    @pl.when(i < NBLK - 1)
    def _():
        sim_ref[...] = sim

    bgm = jnp.max(sim.reshape(Q, GB, R), axis=-1)
    gid = jax.lax.broadcasted_iota(jnp.int32, (Q, GB), 1) + i * GB
    gm_ref[0] = jnp.where(gid < GREAL, bgm, PAD_VAL)


def _bisect_select(gm_ref, gsel_ref):
    gm = gm_ref[...]

    def bis(_, lohi):
        lo, hi = lohi
        mid = 0.5 * (lo + hi)
        ge = (gm >= mid[None, :, None]).astype(jnp.float32)
        cnt = jnp.sum(jnp.sum(ge, axis=-1), axis=0)
        take = cnt >= float(CAND)
        return jnp.where(take, mid, lo), jnp.where(take, hi, mid)

    lo0 = jnp.full((Q,), -1.01, jnp.float32)
    hi0 = jnp.full((Q,), 1.01, jnp.float32)
    tau, _ = jax.lax.fori_loop(0, BISECT_ITERS, bis, (lo0, hi0))

    gid3 = (jax.lax.broadcasted_iota(jnp.int32, (NBLK, Q, GB), 0) * GB
            + jax.lax.broadcasted_iota(jnp.int32, (NBLK, Q, GB), 2))
    work = jnp.where(gm >= tau[None, :, None], gid3, BIG)
    ocol = jax.lax.broadcasted_iota(jnp.int32, (Q, GSEL), 1)

    def ext(t, carry):
        work, out = carry
        p = jnp.min(jnp.min(work, axis=-1), axis=0)
        work = jnp.where(work == p[None, :, None], BIG, work)
        out = jnp.where(ocol == t, jnp.minimum(p, G - 1)[:, None], out)
        return work, out

    _, gsel = jax.lax.fori_loop(
        0, GSEL, ext, (work, jnp.zeros((Q, GSEL), jnp.int32)))
    gsel_ref[...] = gsel


def _expand(gsel_ref, ridx_ref, sub_ref, civ_ref):
    gsel = gsel_ref[...]
    qrow = jax.lax.broadcasted_iota(jnp.int32, (Q, GSEL), 0)
    ridx_ref[...] = qrow * G4 + gsel // 4
    sub = gsel % 4
    sub_ref[...] = jnp.broadcast_to(
        sub[:, :, None], (Q, GSEL, 16)).reshape(Q, GSEL * 16)
    c32 = jax.lax.broadcasted_iota(jnp.int32, (Q, GSEL, R), 2)
    civ_ref[...] = (gsel[:, :, None] * R + c32).reshape(Q, CW)


def _extract_topk(vals_ref, idx_ref, ovals_ref, oidx_ref):
    v = vals_ref[...]
    ci = idx_ref[...]
    ocol = jax.lax.broadcasted_iota(jnp.int32, (Q, CAND), 1)

    def body(t, carry):
        v, ov, oi = carry
        m = jnp.max(v, axis=1, keepdims=True)
        g = jnp.min(jnp.where(v == m, ci, BIG), axis=1, keepdims=True)
        v = jnp.where(ci == g, FILL_VAL, v)
        ov = jnp.where(ocol == t, m, ov)
        oi = jnp.where(ocol == t, g, oi)
        return v, ov, oi

    ov0 = jnp.zeros((Q, CAND), jnp.float32)
    oi0 = jnp.zeros((Q, CAND), jnp.int32)
    _, ov, oi = jax.lax.fori_loop(0, CAND, body, (v, ov0, oi0))
    ovals_ref[...] = ov
    oidx_ref[...] = oi


NWORK = 32
QW = Q // NWORK


def _sc_gather_body(ridx_hbm, sub_hbm, sim4_hbm, cvals_hbm,
                    ridx_v, sub_v, rows_v, cv_v, sem):
    wid = jax.lax.axis_index("s") * 2 + jax.lax.axis_index("c")
    q0 = wid * QW

    def per_query(ql, carry):
        q = q0 + ql
        pltpu.sync_copy(ridx_hbm.at[q], ridx_v)
        pltpu.sync_copy(sub_hbm.at[q], sub_v)
        pltpu.async_copy(sim4_hbm.at[ridx_v], rows_v, sem).wait()
        for j in range(GSEL):
            s = sub_v[pl.ds(j * 16, 16)]
            e0 = s == 0
            e1 = s == 1
            e2 = s == 2
            r0 = rows_v[j, pl.ds(0, 16)]
            r2 = rows_v[j, pl.ds(32, 16)]
            r4 = rows_v[j, pl.ds(64, 16)]
            r6 = rows_v[j, pl.ds(96, 16)]
            lo = jnp.where(e0, r0, jnp.where(e1, r2, jnp.where(e2, r4, r6)))
            r1 = rows_v[j, pl.ds(16, 16)]
            r3 = rows_v[j, pl.ds(48, 16)]
            r5 = rows_v[j, pl.ds(80, 16)]
            r7 = rows_v[j, pl.ds(112, 16)]
            hi = jnp.where(e0, r1, jnp.where(e1, r3, jnp.where(e2, r5, r7)))
            cv_v[pl.ds(j * R, 16)] = lo
            cv_v[pl.ds(j * R + 16, 16)] = hi
        pltpu.sync_copy(cv_v, cvals_hbm.at[q])
        return carry

    jax.lax.fori_loop(0, QW, per_query, jnp.int32(0))


def _stage_b_sc(ridx, sub, sim4):
    mesh = plsc.VectorSubcoreMesh(core_axis_name="c", subcore_axis_name="s")
    run = functools.partial(
        pl.kernel,
        mesh=mesh,
        out_type=jax.ShapeDtypeStruct((Q, CW), jnp.float32),
        scratch_types=[
            pltpu.VMEM((GSEL,), jnp.int32),
            pltpu.VMEM((GSEL * 16,), jnp.int32),
            pltpu.VMEM((GSEL, 128), jnp.float32),
            pltpu.VMEM((CW,), jnp.float32),
            pltpu.SemaphoreType.DMA,
        ],
    )(_sc_gather_body)
    return run(ridx, sub, sim4)


_INTERPRET = False


def _stage_a(queries, keys_p):
    return pl.pallas_call(
        _simgm_block,
        grid=(NBLK,),
        in_specs=[
            pl.BlockSpec((Q, D), lambda i: (0, 0)),
            pl.BlockSpec((KB, D), lambda i: (i, 0)),
        ],
        out_specs=[
            pl.BlockSpec((Q, KB), lambda i: (0, i)),
            pl.BlockSpec((1, Q, GB), lambda i: (i, 0, 0)),
        ],
        out_shape=[
            jax.ShapeDtypeStruct((Q, KPAD), jnp.float32),
            jax.ShapeDtypeStruct((NBLK, Q, GB), jnp.float32),
        ],
        interpret=_INTERPRET,
    )(queries, keys_p)


def _stage_a2(gm):
    return pl.pallas_call(
        _bisect_select,
        in_specs=[pl.BlockSpec((NBLK, Q, GB), lambda: (0, 0, 0))],
        out_specs=pl.BlockSpec((Q, GSEL), lambda: (0, 0)),
        out_shape=jax.ShapeDtypeStruct((Q, GSEL), jnp.int32),
        interpret=_INTERPRET,
    )(gm)


def _stage_a3(gsel):
    return pl.pallas_call(
        _expand,
        in_specs=[pl.BlockSpec((Q, GSEL), lambda: (0, 0))],
        out_specs=[
            pl.BlockSpec((Q, GSEL), lambda: (0, 0)),
            pl.BlockSpec((Q, GSEL * 16), lambda: (0, 0)),
            pl.BlockSpec((Q, CW), lambda: (0, 0)),
        ],
        out_shape=[
            jax.ShapeDtypeStruct((Q, GSEL), jnp.int32),
            jax.ShapeDtypeStruct((Q, GSEL * 16), jnp.int32),
            jax.ShapeDtypeStruct((Q, CW), jnp.int32),
        ],
        interpret=_INTERPRET,
    )(gsel)


def _stage_d(cvals, cidx):
    return pl.pallas_call(
        _extract_topk,
        in_specs=[
            pl.BlockSpec((Q, CW), lambda: (0, 0)),
            pl.BlockSpec((Q, CW), lambda: (0, 0)),
        ],
        out_specs=[
            pl.BlockSpec((Q, CAND), lambda: (0, 0)),
            pl.BlockSpec((Q, CAND), lambda: (0, 0)),
        ],
        out_shape=[
            jax.ShapeDtypeStruct((Q, CAND), jnp.float32),
            jax.ShapeDtypeStruct((Q, CAND), jnp.int32),
        ],
        interpret=_INTERPRET,
    )(cvals, cidx)


def kernel(queries, keys):
    keys_p = jnp.pad(keys, ((0, KPAD - K), (0, 0)))
    sim, gm = _stage_a(queries, keys_p)
    gsel = _stage_a2(gm)
    ridx, sub, civ = _stage_a3(gsel)
    cvals = _stage_b_sc(ridx, sub, sim.reshape(Q * G4, 128))
    vals, idx = _stage_d(cvals, civ)
    return vals, idx
